# baseline (speedup 1.0000x reference)
import functools

import jax
import jax.numpy as jnp
from jax import lax
from jax.experimental import pallas as pl
from jax.experimental.pallas import tpu as pltpu
from jax.experimental.pallas import tpu_sc as plsc

NUM_CORES = 2
NUM_SUBCORES = 16
NW = NUM_CORES * NUM_SUBCORES


def _sc_slot(table, t):
    d = table.shape[1]
    rows_per_w = t // NW
    chunk = min(rows_per_w, 64)
    n_chunks = rows_per_w // chunk

    mesh = plsc.VectorSubcoreMesh(core_axis_name="c", subcore_axis_name="s")

    @functools.partial(
        pl.kernel,
        mesh=mesh,
        out_type=jax.ShapeDtypeStruct((t, d), jnp.float32),
        scratch_types=[pltpu.VMEM((chunk, d), jnp.float32)],
    )
    def body(table_hbm, out_hbm, buf):
        wid = lax.axis_index("s") * NUM_CORES + lax.axis_index("c")
        base = wid * rows_per_w
        for c in range(n_chunks):
            r0 = base + c * chunk
            pltpu.sync_copy(table_hbm.at[pl.ds(r0, chunk)], buf)
            pltpu.sync_copy(buf, out_hbm.at[pl.ds(r0, chunk)])

    return body(table)


def _tc_broadcast(tslice, b, t):
    d = tslice.shape[1]
    bt = 512
    nb = b - 1

    def body(tab_ref, out_ref):
        out_ref[...] = jnp.broadcast_to(tab_ref[...][None], (nb, bt, d))

    return pl.pallas_call(
        body,
        grid=(t // bt,),
        in_specs=[pl.BlockSpec((bt, d), lambda j: (j, 0))],
        out_specs=pl.BlockSpec((nb, bt, d), lambda j: (0, j, 0)),
        out_shape=jax.ShapeDtypeStruct((b, t, d), jnp.float32),
    )(tslice)


def _tc_stitch(big, sc_part, b, t):
    d = big.shape[2]
    bt = 512

    def body(big_ref, sc_ref, out_ref):
        out_ref[...] = sc_ref[...][None]

    return pl.pallas_call(
        body,
        grid=(t // bt,),
        in_specs=[
            pl.BlockSpec(memory_space=pl.ANY),
            pl.BlockSpec((bt, d), lambda j: (j, 0)),
        ],
        out_specs=pl.BlockSpec((1, bt, d), lambda j: (b - 1, j, 0)),
        out_shape=jax.ShapeDtypeStruct((b, t, d), jnp.float32),
        input_output_aliases={0: 0},
    )(big, sc_part)


@functools.partial(jax.jit, static_argnums=(1, 2))
def _posemb(table, b, t):
    tslice = lax.optimization_barrier(table[:t] * jnp.float32(1.0))
    sc_part = _sc_slot(table, t)
    big = _tc_broadcast(tslice, b, t)
    return _tc_stitch(big, sc_part, b, t)


def kernel(x, positional_emb):
    b, t = x.shape
    return _posemb(positional_emb, b, t)


# speedup vs baseline: 1.6011x; 1.6011x over previous
import functools

import jax
import jax.numpy as jnp
from jax import lax
from jax.experimental import pallas as pl
from jax.experimental.pallas import tpu as pltpu
from jax.experimental.pallas import tpu_sc as plsc

NUM_CORES = 2
NUM_SUBCORES = 16
NW = NUM_CORES * NUM_SUBCORES


@functools.partial(jax.jit, static_argnums=(1, 2))
def _posemb_sc(table, b, t):
    d = table.shape[1]
    rows_per_w = t // NW
    chunk = min(rows_per_w, 64)
    n_chunks = rows_per_w // chunk

    mesh = plsc.VectorSubcoreMesh(core_axis_name="c", subcore_axis_name="s")

    @functools.partial(
        pl.kernel,
        mesh=mesh,
        out_type=jax.ShapeDtypeStruct((b * t, d), jnp.float32),
        scratch_types=[
            pltpu.VMEM((chunk, d), jnp.float32),
        ],
    )
    def body(table_hbm, out_hbm, buf):
        wid = lax.axis_index("s") * NUM_CORES + lax.axis_index("c")
        base = wid * rows_per_w
        for c in range(n_chunks):
            r0 = base + c * chunk
            pltpu.sync_copy(table_hbm.at[pl.ds(r0, chunk)], buf)
            for bi in range(b):
                pltpu.sync_copy(buf, out_hbm.at[pl.ds(bi * t + r0, chunk)])

    return body(table)


def kernel(x, positional_emb):
    b, t = x.shape
    assert t % NW == 0
    out = _posemb_sc(positional_emb, b, t)
    return out.reshape(b, t, positional_emb.shape[1])
